# initial kernel scaffold (unmeasured)
import jax
import jax.numpy as jnp
from jax import lax
from jax.experimental import pallas as pl
from jax.experimental.pallas import tpu as pltpu

N_DEV = 16
M = 4096
N_OUT = 2048
CHUNK = M // N_DEV
COMM_DTYPE = jnp.bfloat16

MESH = pl.DeviceIdType.MESH
N_STEPS = 2 * (N_DEV - 1)


def _body(x_ref, w_ref, sx_ref, sw_ref, out_ref,
          comm_ref, send_sems, recv_sems, credit_sem):
    my = lax.axis_index("i")
    left = (my - 1) % N_DEV
    right = (my + 1) % N_DEV

    barrier = pltpu.get_barrier_semaphore()
    for nbr in (left, right):
        pl.semaphore_signal(barrier, inc=1, device_id=(nbr,),
                            device_id_type=MESH)
    pl.semaphore_wait(barrier, 2)

    scale = sx_ref[0] * sw_ref[0]
    partial = lax.dot_general(
        x_ref[:, :].astype(jnp.bfloat16),
        w_ref[:, :].astype(jnp.bfloat16),
        (((1,), (0,)), ((), ())),
        preferred_element_type=jnp.float32,
    )
    out_ref[:, :] = partial * scale

    def rows(c):
        return (pl.ds(c * CHUNK, CHUNK), slice(None))

    for g in range(N_STEPS):
        slot_s = g % 2
        slot_r = (g + 1) % 2
        if g == 0:
            comm_ref[0, :, :] = out_ref[rows(my)].astype(COMM_DTYPE)
        if g >= 1:
            pl.semaphore_wait(credit_sem, 1)
        rdma = pltpu.make_async_remote_copy(
            src_ref=comm_ref.at[slot_s],
            dst_ref=comm_ref.at[slot_r],
            send_sem=send_sems.at[slot_s],
            recv_sem=recv_sems.at[slot_r],
            device_id=(right,),
            device_id_type=MESH,
        )
        rdma.start()
        rdma.wait()
        if g < N_STEPS - 1:
            pl.semaphore_signal(credit_sem, inc=1, device_id=(left,),
                                device_id_type=MESH)
        if g < N_DEV - 1:
            c = (my - g - 1) % N_DEV
            acc = comm_ref[slot_r, :, :].astype(jnp.float32) + out_ref[rows(c)]
            if g == N_DEV - 2:
                out_ref[rows(c)] = acc
            comm_ref[slot_r, :, :] = acc.astype(COMM_DTYPE)
        else:
            t = g - (N_DEV - 1)
            c = (my - t) % N_DEV
            out_ref[rows(c)] = comm_ref[slot_r, :, :].astype(jnp.float32)


def kernel(x, w_mat, scale_x, scale_w):
    return pl.pallas_call(
        _body,
        out_shape=jax.ShapeDtypeStruct((M, N_OUT), jnp.float32),
        in_specs=[
            pl.BlockSpec(memory_space=pltpu.VMEM),
            pl.BlockSpec(memory_space=pltpu.VMEM),
            pl.BlockSpec(memory_space=pltpu.SMEM),
            pl.BlockSpec(memory_space=pltpu.SMEM),
        ],
        out_specs=pl.BlockSpec(memory_space=pltpu.VMEM),
        scratch_shapes=[
            pltpu.VMEM((2, CHUNK, N_OUT), COMM_DTYPE),
            pltpu.SemaphoreType.DMA((2,)),
            pltpu.SemaphoreType.DMA((2,)),
            pltpu.SemaphoreType.REGULAR,
        ],
        compiler_params=pltpu.CompilerParams(collective_id=0),
    )(x, w_mat, scale_x, scale_w)


# baseline (device time: 536173 ns/iter reference)
import jax
import jax.numpy as jnp
from jax import lax
from jax.experimental import pallas as pl
from jax.experimental.pallas import tpu as pltpu

N_DEV = 16
M = 4096
N_OUT = 2048
CHUNK = M // N_DEV
COMM_DTYPE = jnp.bfloat16

MESH = pl.DeviceIdType.MESH
N_STEPS = 2 * (N_DEV - 1)


def _body(x_ref, w_ref, sx_ref, sw_ref, out_ref,
          comm_ref, send_sems, recv_sems, credit_sem):
    my = lax.axis_index("i")
    left = (my - 1) % N_DEV
    right = (my + 1) % N_DEV

    barrier = pltpu.get_barrier_semaphore()
    for nbr in (left, right):
        pl.semaphore_signal(barrier, inc=1, device_id=(nbr,),
                            device_id_type=MESH)
    pl.semaphore_wait(barrier, 2)

    scale = sx_ref[0] * sw_ref[0]
    partial = lax.dot_general(
        x_ref[:, :].astype(jnp.bfloat16),
        w_ref[:, :].astype(jnp.bfloat16),
        (((1,), (0,)), ((), ())),
        preferred_element_type=jnp.float32,
    )
    out_ref[:, :] = partial * scale

    def rows(c):
        return (pl.ds(c * CHUNK, CHUNK), slice(None))

    for g in range(N_STEPS):
        slot_s = g % 2
        slot_r = (g + 1) % 2
        if g == 0:
            comm_ref[0, :, :] = out_ref[rows(my)].astype(COMM_DTYPE)
        if g >= 1:
            pl.semaphore_wait(credit_sem, 1)
        rdma = pltpu.make_async_remote_copy(
            src_ref=comm_ref.at[slot_s],
            dst_ref=comm_ref.at[slot_r],
            send_sem=send_sems.at[slot_s],
            recv_sem=recv_sems.at[slot_r],
            device_id=(right,),
            device_id_type=MESH,
        )
        rdma.start()
        rdma.wait()
        if g < N_STEPS - 1:
            pl.semaphore_signal(credit_sem, inc=1, device_id=(left,),
                                device_id_type=MESH)
        if g < N_DEV - 1:
            c = (my - g - 1) % N_DEV
            acc = comm_ref[slot_r, :, :].astype(jnp.float32) + out_ref[rows(c)]
            if g == N_DEV - 2:
                out_ref[rows(c)] = acc
            comm_ref[slot_r, :, :] = acc.astype(COMM_DTYPE)
        else:
            t = g - (N_DEV - 1)
            c = (my - t) % N_DEV
            out_ref[rows(c)] = comm_ref[slot_r, :, :].astype(jnp.float32)


def kernel(x, w_mat, scale_x, scale_w):
    return pl.pallas_call(
        _body,
        out_shape=jax.ShapeDtypeStruct((M, N_OUT), jnp.float32),
        in_specs=[
            pl.BlockSpec(memory_space=pltpu.VMEM),
            pl.BlockSpec(memory_space=pltpu.VMEM),
            pl.BlockSpec(memory_space=pltpu.SMEM),
            pl.BlockSpec(memory_space=pltpu.SMEM),
        ],
        out_specs=pl.BlockSpec(memory_space=pltpu.VMEM),
        scratch_shapes=[
            pltpu.VMEM((2, CHUNK, N_OUT), COMM_DTYPE),
            pltpu.SemaphoreType.DMA((2,)),
            pltpu.SemaphoreType.DMA((2,)),
            pltpu.SemaphoreType.REGULAR,
        ],
        compiler_params=pltpu.CompilerParams(
            collective_id=0,
            vmem_limit_bytes=100 * 1024 * 1024,
        ),
    )(x, w_mat, scale_x, scale_w)


# device time: 367764 ns/iter; 1.4579x vs baseline; 1.4579x over previous
import jax
import jax.numpy as jnp
from jax import lax
from jax.experimental import pallas as pl
from jax.experimental.pallas import tpu as pltpu

N_DEV = 16
M = 4096
N_OUT = 2048
CHUNK = M // N_DEV
HALF = N_OUT // 2
COMM_DTYPE = jnp.bfloat16

MESH = pl.DeviceIdType.MESH
N_STEPS = 2 * (N_DEV - 1)


def _body(x_ref, w_ref, sx_ref, sw_ref, out_ref,
          cw_ref, ccw_ref, cw_send_sems, cw_recv_sems,
          ccw_send_sems, ccw_recv_sems, cw_credit, ccw_credit):
    my = lax.axis_index("i")
    left = (my - 1) % N_DEV
    right = (my + 1) % N_DEV

    barrier = pltpu.get_barrier_semaphore()
    for nbr in (left, right):
        pl.semaphore_signal(barrier, inc=1, device_id=(nbr,),
                            device_id_type=MESH)
    pl.semaphore_wait(barrier, 2)

    scale = sx_ref[0] * sw_ref[0]
    partial = lax.dot_general(
        x_ref[:, :].astype(jnp.bfloat16),
        w_ref[:, :].astype(jnp.bfloat16),
        (((1,), (0,)), ((), ())),
        preferred_element_type=jnp.float32,
    )
    out_ref[:, :] = partial * scale

    def cw_rows(c):
        return (pl.ds(c * CHUNK, CHUNK), pl.ds(0, HALF))

    def ccw_rows(c):
        return (pl.ds(c * CHUNK, CHUNK), pl.ds(HALF, HALF))

    for g in range(N_STEPS):
        slot_s = g % 2
        slot_r = (g + 1) % 2
        if g == 0:
            cw_ref[0, :, :] = out_ref[cw_rows(my)].astype(COMM_DTYPE)
            ccw_ref[0, :, :] = out_ref[ccw_rows(my)].astype(COMM_DTYPE)
        if g >= 1:
            pl.semaphore_wait(cw_credit, 1)
            pl.semaphore_wait(ccw_credit, 1)
        cw = pltpu.make_async_remote_copy(
            src_ref=cw_ref.at[slot_s],
            dst_ref=cw_ref.at[slot_r],
            send_sem=cw_send_sems.at[slot_s],
            recv_sem=cw_recv_sems.at[slot_r],
            device_id=(right,),
            device_id_type=MESH,
        )
        ccw = pltpu.make_async_remote_copy(
            src_ref=ccw_ref.at[slot_s],
            dst_ref=ccw_ref.at[slot_r],
            send_sem=ccw_send_sems.at[slot_s],
            recv_sem=ccw_recv_sems.at[slot_r],
            device_id=(left,),
            device_id_type=MESH,
        )
        cw.start()
        ccw.start()
        cw.wait_recv()
        ccw.wait_recv()
        cw.wait_send()
        ccw.wait_send()
        if g < N_STEPS - 1:
            pl.semaphore_signal(cw_credit, inc=1, device_id=(left,),
                                device_id_type=MESH)
            pl.semaphore_signal(ccw_credit, inc=1, device_id=(right,),
                                device_id_type=MESH)
        if g < N_DEV - 1:
            c_cw = (my - g - 1) % N_DEV
            c_ccw = (my + g + 1) % N_DEV
            acc_cw = (cw_ref[slot_r, :, :].astype(jnp.float32)
                      + out_ref[cw_rows(c_cw)])
            acc_ccw = (ccw_ref[slot_r, :, :].astype(jnp.float32)
                       + out_ref[ccw_rows(c_ccw)])
            if g == N_DEV - 2:
                out_ref[cw_rows(c_cw)] = acc_cw
                out_ref[ccw_rows(c_ccw)] = acc_ccw
            cw_ref[slot_r, :, :] = acc_cw.astype(COMM_DTYPE)
            ccw_ref[slot_r, :, :] = acc_ccw.astype(COMM_DTYPE)
        else:
            t = g - (N_DEV - 1)
            c_cw = (my - t) % N_DEV
            c_ccw = (my + t) % N_DEV
            out_ref[cw_rows(c_cw)] = cw_ref[slot_r, :, :].astype(jnp.float32)
            out_ref[ccw_rows(c_ccw)] = ccw_ref[slot_r, :, :].astype(jnp.float32)


def kernel(x, w_mat, scale_x, scale_w):
    return pl.pallas_call(
        _body,
        out_shape=jax.ShapeDtypeStruct((M, N_OUT), jnp.float32),
        in_specs=[
            pl.BlockSpec(memory_space=pltpu.VMEM),
            pl.BlockSpec(memory_space=pltpu.VMEM),
            pl.BlockSpec(memory_space=pltpu.SMEM),
            pl.BlockSpec(memory_space=pltpu.SMEM),
        ],
        out_specs=pl.BlockSpec(memory_space=pltpu.VMEM),
        scratch_shapes=[
            pltpu.VMEM((2, CHUNK, HALF), COMM_DTYPE),
            pltpu.VMEM((2, CHUNK, HALF), COMM_DTYPE),
            pltpu.SemaphoreType.DMA((2,)),
            pltpu.SemaphoreType.DMA((2,)),
            pltpu.SemaphoreType.DMA((2,)),
            pltpu.SemaphoreType.DMA((2,)),
            pltpu.SemaphoreType.REGULAR,
            pltpu.SemaphoreType.REGULAR,
        ],
        compiler_params=pltpu.CompilerParams(
            collective_id=0,
            vmem_limit_bytes=100 * 1024 * 1024,
        ),
    )(x, w_mat, scale_x, scale_w)
